# baseline (speedup 1.0000x reference)
import functools

import jax
import jax.numpy as jnp
from jax import lax
from jax.experimental import pallas as pl
from jax.experimental.pallas import tpu as pltpu
from jax.experimental.pallas import tpu_sc as plsc

V = 1000000
D = 64
B = 4096
H = 200
NW = 32
SCALE = 8.0

TCOLS = V // 128
TREM = V - TCOLS * 128
KPW1 = TCOLS // NW + 1

_mesh = plsc.VectorSubcoreMesh(core_axis_name="c", subcore_axis_name="s")


def _iota16():
    return lax.iota(jnp.int32, 16)


@functools.partial(
    pl.kernel,
    out_type=jax.ShapeDtypeStruct((V // 2, 128), jnp.float32),
    mesh=_mesh,
    compiler_params=pltpu.CompilerParams(
        use_tc_tiling_on_sc=True, needs_layout_passes=False),
    scratch_types=[
        pltpu.VMEM((2, D, 128), jnp.float32),
        pltpu.VMEM((2, D, 128), jnp.float32),
        pltpu.SemaphoreType.DMA((2,)),
        pltpu.SemaphoreType.DMA((2,)),
    ],
)
def _relayout(tt_hbm, tail_hbm, tl_hbm, ibuf, obuf, isem, osem):
    wid = lax.axis_index("s") * 2 + lax.axis_index("c")

    def cstart(k):
        return (k * NW + wid) * 128

    def load(k, b):
        pltpu.async_copy(
            tt_hbm.at[:, pl.ds(pl.multiple_of(cstart(k), 128), 128)],
            ibuf.at[b], isem.at[b])

    @pl.when(wid == 5)
    def _():
        pltpu.sync_copy(tail_hbm, obuf.at[0, pl.ds(0, 32)])
        pltpu.sync_copy(obuf.at[0, pl.ds(0, 32)],
                        tl_hbm.at[pl.ds(TCOLS * 64, 32)])

    load(0, 0)
    load(1, 1)

    def step(k, b):
        c0 = pl.multiple_of(cstart(k), 128)
        full = c0 + 128 <= V

        def transpose(nk2):
            def rowstep(k2, c2):
                for jg in range(8):
                    dvec = (jg % 4) * 16 + _iota16()
                    col = jnp.broadcast_to(2 * k2 + (1 if jg >= 4 else 0),
                                           (16,))
                    vals = plsc.load_gather(ibuf.at[b], [dvec, col])
                    obuf[b, k2, pl.ds(jg * 16, 16)] = vals * SCALE
                return c2
            lax.fori_loop(0, nk2, rowstep, 0)

        def wait_prev_store():
            @pl.when(k >= 2)
            def _():
                pltpu.make_async_copy(
                    obuf.at[b], tl_hbm.at[pl.ds(0, D)], osem.at[b]).wait()

        @pl.when(full)
        def _():
            pltpu.make_async_copy(
                tt_hbm.at[:, pl.ds(0, 128)], ibuf.at[b], isem.at[b]).wait()
            wait_prev_store()
            transpose(D)
            pltpu.async_copy(
                obuf.at[b],
                tl_hbm.at[pl.ds(pl.multiple_of(c0 // 2, 64), D)], osem.at[b])

        @pl.when(cstart(k + 2) + 128 <= V)
        def _():
            load(k + 2, b)

    def pair(kk, carry):
        step(kk * 2, 0)
        step(kk * 2 + 1, 1)
        return carry

    lax.fori_loop(0, (KPW1 + 1) // 2, pair, 0)

    for b in range(2):
        pltpu.make_async_copy(
            obuf.at[b], tl_hbm.at[pl.ds(0, D)], osem.at[b]).wait()


G = 4


@functools.partial(
    pl.kernel,
    out_type=jax.ShapeDtypeStruct((H, 8, 32, 8, 128), jnp.float32),
    mesh=_mesh,
    compiler_params=pltpu.CompilerParams(
        use_tc_tiling_on_sc=False, needs_layout_passes=False),
    scratch_types=[
        pltpu.VMEM((H, 128), jnp.int32),
        pltpu.VMEM((G, 128, D), jnp.float32),
        pltpu.VMEM((G, 8, 8, 128), jnp.float32),
        pltpu.SemaphoreType.DMA((G,)),
        pltpu.SemaphoreType.DMA((G,)),
    ],
)
def _hgather(xt_hbm, tlin_hbm, out_hbm, idx_v, gbuf, obuf, gsem, osem):
    wid = lax.axis_index("s") * 2 + lax.axis_index("c")
    pltpu.sync_copy(xt_hbm.at[:, pl.ds(wid * 128, 128)], idx_v)

    for g in range(G):
        pltpu.async_copy(
            tlin_hbm.at[idx_v.at[g]], gbuf.at[g], gsem.at[g])

    def outer(ii, carry):
        for g in range(G):
            h = ii * G + g
            pltpu.make_async_copy(
                tlin_hbm.at[idx_v.at[h]], gbuf.at[g], gsem.at[g]).wait()

            @pl.when(ii > 0)
            def _():
                pltpu.make_async_copy(
                    obuf.at[g], out_hbm.at[0, pl.ds(0, 8), 0], osem.at[g]).wait()

            def dtstep(dt, c2):
                for dd in range(8):
                    col = jnp.broadcast_to(dt * 8 + dd, (16,))
                    for bg in range(8):
                        rows = bg * 16 + _iota16()
                        vals = plsc.load_gather(gbuf.at[g], [rows, col])
                        obuf[g, dt, dd, pl.ds(bg * 16, 16)] = vals
                return c2

            lax.fori_loop(0, 8, dtstep, 0)

            pltpu.async_copy(
                obuf.at[g], out_hbm.at[h, pl.ds(0, 8), wid], osem.at[g])

            @pl.when(h + G < H)
            def _():
                pltpu.async_copy(
                    tlin_hbm.at[idx_v.at[h + G]], gbuf.at[g], gsem.at[g])
        return carry

    lax.fori_loop(0, H // G, outer, 0)

    for g in range(G):
        pltpu.make_async_copy(
            obuf.at[g], out_hbm.at[0, pl.ds(0, 8), 0], osem.at[g]).wait()


def kernel(x, table):
    tt = table.T
    tail = (table[V - TREM:] * SCALE).reshape(TREM // 2, 128)
    tl = _relayout(tt, tail)
    tlin = tl.reshape(V, D)
    xt = x.T
    op = _hgather(xt, tlin)
    return op.transpose(2, 4, 0, 1, 3).reshape(B, H, D)


# speedup vs baseline: 1.4159x; 1.4159x over previous
import functools

import jax
import jax.numpy as jnp
from jax import lax
from jax.experimental import pallas as pl
from jax.experimental.pallas import tpu as pltpu
from jax.experimental.pallas import tpu_sc as plsc

V = 1000000
D = 64
B = 4096
H = 200
NW = 32
SCALE = 8.0

TCOLS = V // 128
TREM = V - TCOLS * 128
KPW1 = TCOLS // NW + 1

_mesh = plsc.VectorSubcoreMesh(core_axis_name="c", subcore_axis_name="s")


def _iota16():
    return lax.iota(jnp.int32, 16)


@functools.partial(
    pl.kernel,
    out_type=jax.ShapeDtypeStruct((V // 2, 128), jnp.float32),
    mesh=_mesh,
    compiler_params=pltpu.CompilerParams(
        use_tc_tiling_on_sc=True, needs_layout_passes=False),
    scratch_types=[
        pltpu.VMEM((2, D, 131), jnp.float32),
        pltpu.VMEM((2, D, 128), jnp.float32),
        pltpu.SemaphoreType.DMA((2,)),
        pltpu.SemaphoreType.DMA((2,)),
    ],
)
def _relayout(tt_hbm, tail_hbm, tl_hbm, ibuf, obuf, isem, osem):
    wid = lax.axis_index("s") * 2 + lax.axis_index("c")

    def cstart(k):
        return (k * NW + wid) * 128

    def load(k, b):
        pltpu.async_copy(
            tt_hbm.at[:, pl.ds(pl.multiple_of(cstart(k), 128), 128)],
            ibuf.at[b, :, pl.ds(0, 128)], isem.at[b])

    @pl.when(wid == 5)
    def _():
        pltpu.sync_copy(tail_hbm, obuf.at[0, pl.ds(0, 32)])
        pltpu.sync_copy(obuf.at[0, pl.ds(0, 32)],
                        tl_hbm.at[pl.ds(TCOLS * 64, 32)])

    load(0, 0)
    load(1, 1)

    def step(k, b):
        c0 = pl.multiple_of(cstart(k), 128)
        full = c0 + 128 <= V

        def transpose(nk2):
            def rowstep(k2, c2):
                for jg in range(8):
                    dvec = (jg % 4) * 16 + _iota16()
                    col = jnp.broadcast_to(2 * k2 + (1 if jg >= 4 else 0),
                                           (16,))
                    vals = plsc.load_gather(ibuf.at[b], [dvec, col])
                    obuf[b, k2, pl.ds(jg * 16, 16)] = vals * SCALE
                return c2
            lax.fori_loop(0, nk2, rowstep, 0)

        def wait_prev_store():
            @pl.when(k >= 2)
            def _():
                pltpu.make_async_copy(
                    obuf.at[b], tl_hbm.at[pl.ds(0, D)], osem.at[b]).wait()

        @pl.when(full)
        def _():
            pltpu.make_async_copy(
                tt_hbm.at[:, pl.ds(0, 128)],
                ibuf.at[b, :, pl.ds(0, 128)], isem.at[b]).wait()
            wait_prev_store()
            transpose(D)
            pltpu.async_copy(
                obuf.at[b],
                tl_hbm.at[pl.ds(pl.multiple_of(c0 // 2, 64), D)], osem.at[b])

        @pl.when(cstart(k + 2) + 128 <= V)
        def _():
            load(k + 2, b)

    def pair(kk, carry):
        step(kk * 2, 0)
        step(kk * 2 + 1, 1)
        return carry

    lax.fori_loop(0, (KPW1 + 1) // 2, pair, 0)

    for b in range(2):
        pltpu.make_async_copy(
            obuf.at[b], tl_hbm.at[pl.ds(0, D)], osem.at[b]).wait()


G = 4


@functools.partial(
    pl.kernel,
    out_type=jax.ShapeDtypeStruct((H, 8, 32, 8, 128), jnp.float32),
    mesh=_mesh,
    compiler_params=pltpu.CompilerParams(
        use_tc_tiling_on_sc=False, needs_layout_passes=False),
    scratch_types=[
        pltpu.VMEM((H, 128), jnp.int32),
        pltpu.VMEM((G, 128, D), jnp.float32),
        pltpu.VMEM((G, 8, 8, 129), jnp.float32),
        pltpu.SemaphoreType.DMA((G,)),
        pltpu.SemaphoreType.DMA((G,)),
    ],
)
def _hgather(xt_hbm, tlin_hbm, out_hbm, idx_v, gbuf, obuf, gsem, osem):
    wid = lax.axis_index("s") * 2 + lax.axis_index("c")
    pltpu.sync_copy(xt_hbm.at[:, pl.ds(wid * 128, 128)], idx_v)

    def gdst(g):
        return gbuf.at[g]

    for g in range(G):
        pltpu.async_copy(tlin_hbm.at[idx_v.at[g]], gdst(g), gsem.at[g])

    def outer(ii, carry):
        for g in range(G):
            h = ii * G + g
            pltpu.make_async_copy(
                tlin_hbm.at[idx_v.at[h]], gdst(g), gsem.at[g]).wait()

            @pl.when(ii > 0)
            def _():
                pltpu.make_async_copy(
                    obuf.at[g, :, :, pl.ds(0, 128)],
                    out_hbm.at[0, pl.ds(0, 8), 0], osem.at[g]).wait()

            dts, dds = [], []
            for c in range(4):
                dvec = c * 16 + _iota16()
                dts.append(lax.shift_right_logical(dvec, 3))
                dds.append(lax.bitwise_and(dvec, 7))

            def rstep(r, c2):
                rv = jnp.broadcast_to(r, (16,))
                for c in range(4):
                    vals = gbuf[g, r, pl.ds(c * 16, 16)]
                    plsc.store_scatter(obuf.at[g], [dts[c], dds[c], rv], vals)
                return c2

            lax.fori_loop(0, 128, rstep, 0)

            pltpu.async_copy(
                obuf.at[g, :, :, pl.ds(0, 128)],
                out_hbm.at[h, pl.ds(0, 8), wid], osem.at[g])

            @pl.when(h + G < H)
            def _():
                pltpu.async_copy(
                    tlin_hbm.at[idx_v.at[h + G]], gdst(g), gsem.at[g])
        return carry

    lax.fori_loop(0, H // G, outer, 0)

    for g in range(G):
        pltpu.make_async_copy(
            obuf.at[g, :, :, pl.ds(0, 128)],
            out_hbm.at[0, pl.ds(0, 8), 0], osem.at[g]).wait()


def kernel(x, table):
    tt = table.T
    tail = (table[V - TREM:] * SCALE).reshape(TREM // 2, 128)
    tl = _relayout(tt, tail)
    tlin = tl.reshape(V, D)
    xt = x.T
    op = _hgather(xt, tlin)
    return op.transpose(2, 4, 0, 1, 3).reshape(B, H, D)


# speedup vs baseline: 1.6225x; 1.1460x over previous
import functools

import jax
import jax.numpy as jnp
from jax import lax
from jax.experimental import pallas as pl
from jax.experimental.pallas import tpu as pltpu
from jax.experimental.pallas import tpu_sc as plsc

V = 1000000
D = 64
B = 4096
H = 200
NW = 32
SCALE = 8.0

TCOLS = V // 128
TREM = V - TCOLS * 128
KPW1 = TCOLS // NW + 1

_mesh = plsc.VectorSubcoreMesh(core_axis_name="c", subcore_axis_name="s")


def _iota16():
    return lax.iota(jnp.int32, 16)


@functools.partial(
    pl.kernel,
    out_type=jax.ShapeDtypeStruct((V // 2, 128), jnp.float32),
    mesh=_mesh,
    compiler_params=pltpu.CompilerParams(
        use_tc_tiling_on_sc=True, needs_layout_passes=False),
    scratch_types=[
        pltpu.VMEM((2, D, 128), jnp.float32),
        pltpu.VMEM((2, D, 136), jnp.float32),
        pltpu.SemaphoreType.DMA((2,)),
        pltpu.SemaphoreType.DMA((2,)),
    ],
)
def _relayout(tt_hbm, tail_hbm, tl_hbm, ibuf, obuf, isem, osem):
    wid = lax.axis_index("s") * 2 + lax.axis_index("c")

    def cstart(k):
        return (k * NW + wid) * 128

    def load(k, b):
        pltpu.async_copy(
            tt_hbm.at[:, pl.ds(pl.multiple_of(cstart(k), 128), 128)],
            ibuf.at[b], isem.at[b])

    @pl.when(wid == 5)
    def _():
        pltpu.sync_copy(tail_hbm, obuf.at[0, pl.ds(0, 32), pl.ds(0, 128)])
        pltpu.sync_copy(obuf.at[0, pl.ds(0, 32), pl.ds(0, 128)],
                        tl_hbm.at[pl.ds(TCOLS * 64, 32)])

    load(0, 0)
    load(1, 1)

    def step(k, b):
        c0 = pl.multiple_of(cstart(k), 128)
        full = c0 + 128 <= V

        def transpose(ncols):
            kvecs, jbase = [], []
            for cc in range(ncols // 16):
                cvec = cc * 16 + _iota16()
                kvecs.append(lax.shift_right_logical(cvec, 1))
                jbase.append(lax.bitwise_and(cvec, 1) * D)

            def rowstep(d, c2):
                dv = jnp.broadcast_to(d, (16,))
                for cc in range(ncols // 16):
                    vals = ibuf[b, d, pl.ds(cc * 16, 16)]
                    plsc.store_scatter(
                        obuf.at[b], [kvecs[cc], jbase[cc] + dv],
                        vals * SCALE)
                return c2
            lax.fori_loop(0, D, rowstep, 0)

        def wait_prev_store():
            @pl.when(k >= 2)
            def _():
                pltpu.make_async_copy(
                    obuf.at[b, :, pl.ds(0, 128)],
                    tl_hbm.at[pl.ds(0, D)], osem.at[b]).wait()

        @pl.when(full)
        def _():
            pltpu.make_async_copy(
                tt_hbm.at[:, pl.ds(0, 128)],
                ibuf.at[b, :, pl.ds(0, 128)], isem.at[b]).wait()
            wait_prev_store()
            transpose(128)
            pltpu.async_copy(
                obuf.at[b, :, pl.ds(0, 128)],
                tl_hbm.at[pl.ds(pl.multiple_of(c0 // 2, 64), D)], osem.at[b])

        @pl.when(cstart(k + 2) + 128 <= V)
        def _():
            load(k + 2, b)

    def pair(kk, carry):
        step(kk * 2, 0)
        step(kk * 2 + 1, 1)
        return carry

    lax.fori_loop(0, (KPW1 + 1) // 2, pair, 0)

    for b in range(2):
        pltpu.make_async_copy(
            obuf.at[b, :, pl.ds(0, 128)],
            tl_hbm.at[pl.ds(0, D)], osem.at[b]).wait()


G = 4


@functools.partial(
    pl.kernel,
    out_type=jax.ShapeDtypeStruct((H, 8, 32, 8, 128), jnp.float32),
    mesh=_mesh,
    compiler_params=pltpu.CompilerParams(
        use_tc_tiling_on_sc=False, needs_layout_passes=False),
    scratch_types=[
        pltpu.VMEM((H, 128), jnp.int32),
        pltpu.VMEM((G, 128, D), jnp.float32),
        pltpu.VMEM((G, 8, 8, 136), jnp.float32),
        pltpu.SemaphoreType.DMA((G,)),
        pltpu.SemaphoreType.DMA((G,)),
    ],
)
def _hgather(xt_hbm, tlin_hbm, out_hbm, idx_v, gbuf, obuf, gsem, osem):
    wid = lax.axis_index("s") * 2 + lax.axis_index("c")
    pltpu.sync_copy(xt_hbm.at[:, pl.ds(wid * 128, 128)], idx_v)

    def gdst(g):
        return gbuf.at[g]

    for g in range(G):
        pltpu.async_copy(tlin_hbm.at[idx_v.at[g]], gdst(g), gsem.at[g])

    def outer(ii, carry):
        for g in range(G):
            h = ii * G + g
            pltpu.make_async_copy(
                tlin_hbm.at[idx_v.at[h]], gdst(g), gsem.at[g]).wait()

            @pl.when(ii > 0)
            def _():
                pltpu.make_async_copy(
                    obuf.at[g, :, :, pl.ds(0, 128)],
                    out_hbm.at[0, pl.ds(0, 8), 0], osem.at[g]).wait()

            dts, dds = [], []
            for c in range(4):
                dvec = c * 16 + _iota16()
                dts.append(lax.shift_right_logical(dvec, 3))
                dds.append(lax.bitwise_and(dvec, 7))

            def rstep(r, c2):
                rv = jnp.broadcast_to(r, (16,))
                for c in range(4):
                    vals = gbuf[g, r, pl.ds(c * 16, 16)]
                    plsc.store_scatter(obuf.at[g], [dts[c], dds[c], rv], vals)
                return c2

            lax.fori_loop(0, 128, rstep, 0)

            pltpu.async_copy(
                obuf.at[g, :, :, pl.ds(0, 128)],
                out_hbm.at[h, pl.ds(0, 8), wid], osem.at[g])

            @pl.when(h + G < H)
            def _():
                pltpu.async_copy(
                    tlin_hbm.at[idx_v.at[h + G]], gdst(g), gsem.at[g])
        return carry

    lax.fori_loop(0, H // G, outer, 0)

    for g in range(G):
        pltpu.make_async_copy(
            obuf.at[g, :, :, pl.ds(0, 128)],
            out_hbm.at[0, pl.ds(0, 8), 0], osem.at[g]).wait()


def kernel(x, table):
    tt = table.T
    tail = (table[V - TREM:] * SCALE).reshape(TREM // 2, 128)
    tl = _relayout(tt, tail)
    tlin = tl.reshape(V, D)
    xt = x.T
    op = _hgather(xt, tlin)
    return op.transpose(2, 4, 0, 1, 3).reshape(B, H, D)


# speedup vs baseline: 2.3045x; 1.4203x over previous
import functools

import jax
import jax.numpy as jnp
from jax import lax
from jax.experimental import pallas as pl
from jax.experimental.pallas import tpu as pltpu
from jax.experimental.pallas import tpu_sc as plsc

D = 64
N = 4096 * 200
LPC = 128
NW = 32
CPW = N // (LPC * NW)
SCALE = 8.0

_mesh = plsc.VectorSubcoreMesh(core_axis_name="c", subcore_axis_name="s")


@functools.partial(
    pl.kernel,
    out_type=jax.ShapeDtypeStruct((N, D), jnp.float32),
    mesh=_mesh,
    compiler_params=pltpu.CompilerParams(use_tc_tiling_on_sc=False),
    scratch_types=[
        pltpu.VMEM((CPW, LPC), jnp.int32),
        pltpu.VMEM((2, LPC, D), jnp.float32),
        pltpu.VMEM((2, LPC, D), jnp.float32),
        pltpu.SemaphoreType.DMA((2,)),
        pltpu.SemaphoreType.DMA((2,)),
    ],
)
def _emb_lookup(x_hbm, table_hbm, out_hbm, idx_v, gbuf, obuf, gsem, osem):
    wid = lax.axis_index("s") * 2 + lax.axis_index("c")
    pltpu.sync_copy(x_hbm.at[pl.ds(wid * CPW, CPW)], idx_v)

    for b in range(2):
        pltpu.async_copy(
            table_hbm.at[idx_v.at[b]], gbuf.at[b], gsem.at[b])

    def chunk(i, b):
        pltpu.make_async_copy(
            table_hbm.at[idx_v.at[i]], gbuf.at[b], gsem.at[b]).wait()

        @pl.when(i >= 2)
        def _():
            pltpu.make_async_copy(
                obuf.at[b], out_hbm.at[pl.ds(0, LPC)], osem.at[b]).wait()

        def srow(r, c2):
            for cc in range(D // 16):
                sl = pl.ds(cc * 16, 16)
                obuf[b, r, sl] = gbuf[b, r, sl] * SCALE
            return c2

        lax.fori_loop(0, LPC, srow, 0)

        base = (wid * CPW + i) * LPC
        pltpu.async_copy(obuf.at[b], out_hbm.at[pl.ds(base, LPC)], osem.at[b])

        @pl.when(i + 2 < CPW)
        def _():
            pltpu.async_copy(
                table_hbm.at[idx_v.at[i + 2]], gbuf.at[b], gsem.at[b])

    def pair(kk, carry):
        chunk(kk * 2, 0)
        chunk(kk * 2 + 1, 1)
        return carry

    lax.fori_loop(0, CPW // 2, pair, 0)

    for b in range(2):
        pltpu.make_async_copy(
            obuf.at[b], out_hbm.at[pl.ds(0, LPC)], osem.at[b]).wait()


def kernel(x, table):
    x2 = x.reshape(N // LPC, LPC)
    out = _emb_lookup(x2, table)
    return out.reshape(4096, 200, D)


# speedup vs baseline: 2.5759x; 1.1178x over previous
import functools

import jax
import jax.numpy as jnp
from jax import lax
from jax.experimental import pallas as pl
from jax.experimental.pallas import tpu as pltpu
from jax.experimental.pallas import tpu_sc as plsc

V = 1000000
D = 64
B = 4096
H = 200
NW = 32
SCALE = 8.0
G = 4

_mesh = plsc.VectorSubcoreMesh(core_axis_name="c", subcore_axis_name="s")


def _iota16():
    return lax.iota(jnp.int32, 16)


@functools.partial(
    pl.kernel,
    out_type=jax.ShapeDtypeStruct((H, 8, 32, 8, 128), jnp.float32),
    mesh=_mesh,
    compiler_params=pltpu.CompilerParams(
        use_tc_tiling_on_sc=False, needs_layout_passes=False),
    scratch_types=[
        pltpu.VMEM((H, 128), jnp.int32),
        pltpu.VMEM((G, 128, D), jnp.float32),
        pltpu.VMEM((G, 8, 8, 136), jnp.float32),
        pltpu.SemaphoreType.DMA((G,)),
        pltpu.SemaphoreType.DMA((G,)),
    ],
)
def _hgather(xt_hbm, table_hbm, out_hbm, idx_v, gbuf, obuf, gsem, osem):
    wid = lax.axis_index("s") * 2 + lax.axis_index("c")
    pltpu.sync_copy(xt_hbm.at[:, pl.ds(wid * 128, 128)], idx_v)

    for g in range(G):
        pltpu.async_copy(table_hbm.at[idx_v.at[g]], gbuf.at[g], gsem.at[g])

    def outer(ii, carry):
        for g in range(G):
            h = ii * G + g
            pltpu.make_async_copy(
                table_hbm.at[idx_v.at[h]], gbuf.at[g], gsem.at[g]).wait()

            @pl.when(ii > 0)
            def _():
                pltpu.make_async_copy(
                    obuf.at[g, :, :, pl.ds(0, 128)],
                    out_hbm.at[0, pl.ds(0, 8), 0], osem.at[g]).wait()

            dts, dds = [], []
            for c in range(4):
                dvec = c * 16 + _iota16()
                dts.append(lax.shift_right_logical(dvec, 3))
                dds.append(lax.bitwise_and(dvec, 7))

            def rstep(r, c2):
                rv = jnp.broadcast_to(r, (16,))
                for c in range(4):
                    vals = gbuf[g, r, pl.ds(c * 16, 16)]
                    plsc.store_scatter(
                        obuf.at[g], [dts[c], dds[c], rv], vals * SCALE)
                return c2

            lax.fori_loop(0, 128, rstep, 0)

            pltpu.async_copy(
                obuf.at[g, :, :, pl.ds(0, 128)],
                out_hbm.at[h, pl.ds(0, 8), wid], osem.at[g])

            @pl.when(h + G < H)
            def _():
                pltpu.async_copy(
                    table_hbm.at[idx_v.at[h + G]], gbuf.at[g], gsem.at[g])
        return carry

    lax.fori_loop(0, H // G, outer, 0)

    for g in range(G):
        pltpu.make_async_copy(
            obuf.at[g, :, :, pl.ds(0, 128)],
            out_hbm.at[0, pl.ds(0, 8), 0], osem.at[g]).wait()


def kernel(x, table):
    xt = x.T
    op = _hgather(xt, table)
    return op.transpose(2, 4, 0, 1, 3).reshape(B, H, D)
